# baseline (speedup 1.0000x reference)
import jax
import jax.numpy as jnp
from jax.experimental import pallas as pl
from jax.experimental.pallas import tpu as pltpu

_INTERPRET = False

_SENT = -3.0e38


def _scores_body(nf_ref, noise_ref, w_ref, b_ref, m_ref,
                 s16_ref, minv_ref, c0_ref, u_ref, ru_ref, d_ref):
    n = nf_ref.shape[1]
    l = 128
    x = nf_ref[0] + 0.05 * noise_ref[0]
    scol = jax.lax.dot_general(
        x, w_ref[...], (((1,), (0,)), ((), ())),
        preferred_element_type=jnp.float32) + b_ref[0, 0]

    mn = jnp.min(scol)
    c0 = (jnp.max(scol) + mn) * 0.5
    minv_ref[...] = mn.reshape(1, 1, 1)
    c0_ref[...] = c0.reshape(1, 1, 1)

    mf = (m_ref[0] != 0).astype(jnp.float32)
    eb = jnp.exp(scol - c0)
    u_ref[0] = mf * eb
    ru_ref[0] = mf / eb
    d_ref[0] = 1.0 - mf

    smcol = jnp.where(mf > 0.5, scol, _SENT)
    eye128 = (jax.lax.broadcasted_iota(jnp.int32, (l, l), 0) ==
              jax.lax.broadcasted_iota(jnp.int32, (l, l), 1)
              ).astype(jnp.float32)
    rows = []
    for r in range(n // l):
        smr = jax.lax.slice(smcol, (r * l, 0), ((r + 1) * l, 1))
        rows.append(jax.lax.dot_general(
            smr, eye128, (((0,), (0,)), ((), ())),
            preferred_element_type=jnp.float32))
    s16_ref[0] = jnp.concatenate(rows, axis=0)


def _bitonic_sort_desc(x):
    _, r, l = x.shape
    n = r * l
    riota = jax.lax.broadcasted_iota(jnp.int32, x.shape, 1)
    ciota = jax.lax.broadcasted_iota(jnp.int32, x.shape, 2)
    k = 2
    while k <= n:
        asc = ((riota * l + ciota) & k) != 0 if k < n else (ciota < 0)
        d = k // 2
        while d >= 1:
            if d < l:
                bit = (ciota & d) != 0
                vp = jnp.where(bit, jnp.roll(x, d, axis=2),
                               jnp.roll(x, -d, axis=2))
            else:
                e = d // l
                bit = (riota & e) != 0
                vp = jnp.where(bit, jnp.roll(x, e, axis=1),
                               jnp.roll(x, -e, axis=1))
            want_min = asc != bit
            x = jnp.where(want_min, jnp.minimum(x, vp), jnp.maximum(x, vp))
            d //= 2
        k *= 2
    return x


def _prefix_sum_2d(x, riota, ciota):
    bb, r, l = x.shape
    d = 1
    while d < l:
        x = x + jnp.where(ciota >= d, jnp.roll(x, d, axis=2), 0.0)
        d *= 2
    rowtot = jax.lax.slice(x, (0, 0, l - 1), (bb, r, l))
    t = rowtot
    e = 1
    while e < r:
        t = t + jnp.where(jax.lax.slice(riota, (0, 0, 0), (bb, r, 1)) >= e,
                          jnp.roll(t, e, axis=1), 0.0)
        e *= 2
    return x + (t - rowtot)


def _make_perm_body(nib, ibk, l):
    def _perm_body(s16_ref, minv_ref, c0_ref, u_ref, ru_ref, d_ref,
                   out_ref, v_scr, rv_scr):
        i = pl.program_id(0)
        n = out_ref.shape[1]
        nq = n // ibk

        @pl.when(i == 0)
        def _():
            fill = jnp.min(minv_ref[...]) - 1.0
            s = jnp.where(s16_ref[...] < -1.0e38, fill, s16_ref[...])
            ss = _bitonic_sort_desc(s)
            c0 = c0_ref[...]
            bv = jnp.exp(ss - c0)
            av = 1.0 / bv
            riota = jax.lax.broadcasted_iota(jnp.int32, ss.shape, 1)
            ciota = jax.lax.broadcasted_iota(jnp.int32, ss.shape, 2)
            pa = _prefix_sum_2d(av, riota, ciota)
            pb = _prefix_sum_2d(bv, riota, ciota)
            bt = jnp.sum(bv, axis=(1, 2), keepdims=True)
            denom = bv * pa + av * (bt - pb)
            rd = 1.0 / denom
            v_scr[...] = bv * rd
            rv_scr[...] = av * rd

        @pl.when(i > 0)
        def _():
            t = i - 1
            b = t // nib
            ib = t % nib
            rpb = ibk // l
            v4 = v_scr[pl.ds(b, 1), pl.ds(ib * rpb, rpb), :][0]
            rv4 = rv_scr[pl.ds(b, 1), pl.ds(ib * rpb, rpb), :][0]
            vrow = jnp.concatenate(
                [jax.lax.slice(v4, (tt, 0), (tt + 1, l)) for tt in range(rpb)],
                axis=1)
            rvrow = jnp.concatenate(
                [jax.lax.slice(rv4, (tt, 0), (tt + 1, l)) for tt in range(rpb)],
                axis=1)
            for q in range(nq):
                sl = pl.ds(q * ibk, ibk)

                @pl.when(ib == q)
                def _():
                    eye = (jax.lax.broadcasted_iota(jnp.int32, (ibk, ibk), 0) ==
                           jax.lax.broadcasted_iota(jnp.int32, (ibk, ibk), 1))
                    p = jnp.minimum(u_ref[0, sl, :] * rvrow,
                                    ru_ref[0, sl, :] * vrow)
                    out_ref[0, sl, :] = jnp.where(eye, p + d_ref[0, sl, :], p)

                @pl.when(ib != q)
                def _():
                    out_ref[0, sl, :] = jnp.minimum(u_ref[0, sl, :] * rvrow,
                                                    ru_ref[0, sl, :] * vrow)
    return _perm_body


def kernel(node_features, mask, W, b, noise):
    B, N, D = node_features.shape
    R, L = N // 128, 128
    mask_col = mask.astype(jnp.int32).reshape(B, N, 1)
    b2 = b.reshape(1, 1)

    fcol = pl.BlockSpec((1, N, 1), lambda i: (i, 0, 0))
    fsc = pl.BlockSpec((1, 1, 1), lambda i: (i, 0, 0))
    ocol = jax.ShapeDtypeStruct((B, N, 1), jnp.float32)
    osc = jax.ShapeDtypeStruct((B, 1, 1), jnp.float32)

    s16, minv, c0v, ucol, rucol, dcol = pl.pallas_call(
        _scores_body,
        grid=(B,),
        in_specs=[
            pl.BlockSpec((1, N, D), lambda i: (i, 0, 0)),
            pl.BlockSpec((1, N, D), lambda i: (i, 0, 0)),
            pl.BlockSpec((D, 1), lambda i: (0, 0)),
            pl.BlockSpec((1, 1), lambda i: (0, 0)),
            fcol,
        ],
        out_specs=[pl.BlockSpec((1, R, L), lambda i: (i, 0, 0)),
                   fsc, fsc, fcol, fcol, fcol],
        out_shape=[jax.ShapeDtypeStruct((B, R, L), jnp.float32),
                   osc, osc, ocol, ocol, ocol],
        interpret=_INTERPRET,
    )(node_features, noise, W, b2, mask_col)

    IBK = 256
    NIB = N // IBK

    def bidx(i):
        return jnp.maximum(i - 1, 0) // NIB

    def ibidx(i):
        return jnp.maximum(i - 1, 0) % NIB

    out = pl.pallas_call(
        _make_perm_body(NIB, IBK, L),
        grid=(1 + B * NIB,),
        in_specs=[
            pl.BlockSpec((B, R, L), lambda i: (0, 0, 0)),
            pl.BlockSpec((B, 1, 1), lambda i: (0, 0, 0)),
            pl.BlockSpec((B, 1, 1), lambda i: (0, 0, 0)),
            pl.BlockSpec((1, N, 1), lambda i: (bidx(i), 0, 0)),
            pl.BlockSpec((1, N, 1), lambda i: (bidx(i), 0, 0)),
            pl.BlockSpec((1, N, 1), lambda i: (bidx(i), 0, 0)),
        ],
        out_specs=pl.BlockSpec((1, N, IBK), lambda i: (bidx(i), 0, ibidx(i))),
        out_shape=jax.ShapeDtypeStruct((B, N, N), jnp.float32),
        scratch_shapes=[
            pltpu.VMEM((B, R, L), jnp.float32),
            pltpu.VMEM((B, R, L), jnp.float32),
        ],
        interpret=_INTERPRET,
    )(s16, minv, c0v, ucol, rucol, dcol)
    return out


# speedup vs baseline: 1.4508x; 1.4508x over previous
import jax
import jax.numpy as jnp
from jax.experimental import pallas as pl
from jax.experimental.pallas import tpu as pltpu

_INTERPRET = False

_SENT = -3.0e38


def _scores_body(nf_ref, noise_ref, w_ref, b_ref, m_ref,
                 s16_ref, minv_ref, c0_ref, u_ref, ru_ref, d_ref):
    n = nf_ref.shape[1]
    l = 128
    x = nf_ref[0] + 0.05 * noise_ref[0]
    scol = jax.lax.dot_general(
        x, w_ref[...], (((1,), (0,)), ((), ())),
        preferred_element_type=jnp.float32) + b_ref[0, 0]

    mn = jnp.min(scol)
    c0 = (jnp.max(scol) + mn) * 0.5
    minv_ref[...] = mn.reshape(1, 1, 1)
    c0_ref[...] = c0.reshape(1, 1, 1)

    mf = (m_ref[0] != 0).astype(jnp.float32)
    eb = jnp.exp(scol - c0)
    u_ref[0] = mf * eb
    ru_ref[0] = mf / eb
    d_ref[0] = 1.0 - mf

    smcol = jnp.where(mf > 0.5, scol, _SENT)
    eye128 = (jax.lax.broadcasted_iota(jnp.int32, (l, l), 0) ==
              jax.lax.broadcasted_iota(jnp.int32, (l, l), 1)
              ).astype(jnp.float32)
    rows = []
    for r in range(n // l):
        smr = jax.lax.slice(smcol, (r * l, 0), ((r + 1) * l, 1))
        rows.append(jax.lax.dot_general(
            smr, eye128, (((0,), (0,)), ((), ())),
            preferred_element_type=jnp.float32))
    s16_ref[0] = jnp.concatenate(rows, axis=0)


def _bitonic_sort_desc(x):
    _, r, l = x.shape
    n = r * l
    riota = jax.lax.broadcasted_iota(jnp.int32, x.shape, 1)
    ciota = jax.lax.broadcasted_iota(jnp.int32, x.shape, 2)
    k = 2
    while k <= n:
        asc = ((riota * l + ciota) & k) != 0 if k < n else (ciota < 0)
        d = k // 2
        while d >= 1:
            if d < l:
                bit = (ciota & d) != 0
                vp = jnp.where(bit, jnp.roll(x, d, axis=2),
                               jnp.roll(x, -d, axis=2))
            else:
                e = d // l
                bit = (riota & e) != 0
                vp = jnp.where(bit, jnp.roll(x, e, axis=1),
                               jnp.roll(x, -e, axis=1))
            want_min = asc != bit
            x = jnp.where(want_min, jnp.minimum(x, vp), jnp.maximum(x, vp))
            d //= 2
        k *= 2
    return x


def _prefix_sum_2d(x, riota, ciota):
    bb, r, l = x.shape
    d = 1
    while d < l:
        x = x + jnp.where(ciota >= d, jnp.roll(x, d, axis=2), 0.0)
        d *= 2
    rowtot = jax.lax.slice(x, (0, 0, l - 1), (bb, r, l))
    t = rowtot
    e = 1
    while e < r:
        t = t + jnp.where(jax.lax.slice(riota, (0, 0, 0), (bb, r, 1)) >= e,
                          jnp.roll(t, e, axis=1), 0.0)
        e *= 2
    return x + (t - rowtot)


def _make_perm_body(nib, ibk, l):
    def _perm_body(s16_ref, minv_ref, c0_ref, u_ref, ru_ref, d_ref,
                   out_ref, v_scr, rv_scr):
        i = pl.program_id(0)
        n = out_ref.shape[1]
        nq = n // ibk

        @pl.when(i == 0)
        def _():
            fill = jnp.min(minv_ref[...]) - 1.0
            s = jnp.where(s16_ref[...] < -1.0e38, fill, s16_ref[...])
            ss = _bitonic_sort_desc(s)
            c0 = c0_ref[...]
            bv = jnp.exp(ss - c0)
            av = 1.0 / bv
            riota = jax.lax.broadcasted_iota(jnp.int32, ss.shape, 1)
            ciota = jax.lax.broadcasted_iota(jnp.int32, ss.shape, 2)
            pa = _prefix_sum_2d(av, riota, ciota)
            pb = _prefix_sum_2d(bv, riota, ciota)
            bt = jnp.sum(bv, axis=(1, 2), keepdims=True)
            denom = bv * pa + av * (bt - pb)
            rd = 1.0 / denom
            v_scr[...] = bv * rd
            rv_scr[...] = av * rd

        @pl.when(i > 0)
        def _():
            t = i - 1
            b = t // nib
            ib = t % nib
            rpb = ibk // l
            v4 = v_scr[pl.ds(b, 1), pl.ds(ib * rpb, rpb), :][0]
            rv4 = rv_scr[pl.ds(b, 1), pl.ds(ib * rpb, rpb), :][0]
            vrow = jnp.concatenate(
                [jax.lax.slice(v4, (tt, 0), (tt + 1, l)) for tt in range(rpb)],
                axis=1)
            rvrow = jnp.concatenate(
                [jax.lax.slice(rv4, (tt, 0), (tt + 1, l)) for tt in range(rpb)],
                axis=1)
            for q in range(nq):
                sl = pl.ds(q * ibk, ibk)

                @pl.when(ib == q)
                def _():
                    eye = (jax.lax.broadcasted_iota(jnp.int32, (ibk, ibk), 0) ==
                           jax.lax.broadcasted_iota(jnp.int32, (ibk, ibk), 1))
                    p = jnp.minimum(u_ref[0, sl, :] * rvrow,
                                    ru_ref[0, sl, :] * vrow)
                    out_ref[0, sl, :] = jnp.where(eye, p + d_ref[0, sl, :], p)

                @pl.when(ib != q)
                def _():
                    out_ref[0, sl, :] = jnp.minimum(u_ref[0, sl, :] * rvrow,
                                                    ru_ref[0, sl, :] * vrow)
    return _perm_body


def kernel(node_features, mask, W, b, noise):
    B, N, D = node_features.shape
    R, L = N // 128, 128
    mask_col = mask.astype(jnp.int32).reshape(B, N, 1)
    b2 = b.reshape(1, 1)

    fcol = pl.BlockSpec((1, N, 1), lambda i: (i, 0, 0))
    fsc = pl.BlockSpec((1, 1, 1), lambda i: (i, 0, 0))
    ocol = jax.ShapeDtypeStruct((B, N, 1), jnp.float32)
    osc = jax.ShapeDtypeStruct((B, 1, 1), jnp.float32)

    s16, minv, c0v, ucol, rucol, dcol = pl.pallas_call(
        _scores_body,
        grid=(B,),
        in_specs=[
            pl.BlockSpec((1, N, D), lambda i: (i, 0, 0)),
            pl.BlockSpec((1, N, D), lambda i: (i, 0, 0)),
            pl.BlockSpec((D, 1), lambda i: (0, 0)),
            pl.BlockSpec((1, 1), lambda i: (0, 0)),
            fcol,
        ],
        out_specs=[pl.BlockSpec((1, R, L), lambda i: (i, 0, 0)),
                   fsc, fsc, fcol, fcol, fcol],
        out_shape=[jax.ShapeDtypeStruct((B, R, L), jnp.float32),
                   osc, osc, ocol, ocol, ocol],
        interpret=_INTERPRET,
    )(node_features, noise, W, b2, mask_col)

    IBK = 1024
    NIB = N // IBK

    def bidx(i):
        return jnp.maximum(i - 1, 0) // NIB

    def ibidx(i):
        return jnp.maximum(i - 1, 0) % NIB

    out = pl.pallas_call(
        _make_perm_body(NIB, IBK, L),
        grid=(1 + B * NIB,),
        in_specs=[
            pl.BlockSpec((B, R, L), lambda i: (0, 0, 0)),
            pl.BlockSpec((B, 1, 1), lambda i: (0, 0, 0)),
            pl.BlockSpec((B, 1, 1), lambda i: (0, 0, 0)),
            pl.BlockSpec((1, N, 1), lambda i: (bidx(i), 0, 0)),
            pl.BlockSpec((1, N, 1), lambda i: (bidx(i), 0, 0)),
            pl.BlockSpec((1, N, 1), lambda i: (bidx(i), 0, 0)),
        ],
        out_specs=pl.BlockSpec((1, N, IBK), lambda i: (bidx(i), 0, ibidx(i))),
        out_shape=jax.ShapeDtypeStruct((B, N, N), jnp.float32),
        scratch_shapes=[
            pltpu.VMEM((B, R, L), jnp.float32),
            pltpu.VMEM((B, R, L), jnp.float32),
        ],
        interpret=_INTERPRET,
    )(s16, minv, c0v, ucol, rucol, dcol)
    return out


# speedup vs baseline: 1.4847x; 1.0234x over previous
import jax
import jax.numpy as jnp
from jax.experimental import pallas as pl
from jax.experimental.pallas import tpu as pltpu

_INTERPRET = False

_SENT = -3.0e38


def _scores_body(nf_ref, noise_ref, w_ref, b_ref, m_ref,
                 s16_ref, minv_ref, c0_ref, u_ref, ru_ref, d_ref):
    n = nf_ref.shape[1]
    l = 128
    x = nf_ref[0] + 0.05 * noise_ref[0]
    scol = jax.lax.dot_general(
        x, w_ref[...], (((1,), (0,)), ((), ())),
        preferred_element_type=jnp.float32) + b_ref[0, 0]

    mn = jnp.min(scol)
    c0 = (jnp.max(scol) + mn) * 0.5
    minv_ref[...] = mn.reshape(1, 1, 1)
    c0_ref[...] = c0.reshape(1, 1, 1)

    mf = (m_ref[0] != 0).astype(jnp.float32)
    eb = jnp.exp(scol - c0)
    u_ref[0] = mf * eb
    ru_ref[0] = mf / eb
    d_ref[0] = 1.0 - mf

    smcol = jnp.where(mf > 0.5, scol, _SENT)
    eye128 = (jax.lax.broadcasted_iota(jnp.int32, (l, l), 0) ==
              jax.lax.broadcasted_iota(jnp.int32, (l, l), 1)
              ).astype(jnp.float32)
    rows = []
    for r in range(n // l):
        smr = jax.lax.slice(smcol, (r * l, 0), ((r + 1) * l, 1))
        rows.append(jax.lax.dot_general(
            smr, eye128, (((0,), (0,)), ((), ())),
            preferred_element_type=jnp.float32))
    s16_ref[0] = jnp.concatenate(rows, axis=0)


def _bitonic_sort_desc(x):
    _, r, l = x.shape
    n = r * l
    riota = jax.lax.broadcasted_iota(jnp.int32, x.shape, 1)
    ciota = jax.lax.broadcasted_iota(jnp.int32, x.shape, 2)
    k = 2
    while k <= n:
        asc = ((riota * l + ciota) & k) != 0 if k < n else (ciota < 0)
        d = k // 2
        while d >= 1:
            if d < l:
                bit = (ciota & d) != 0
                vp = jnp.where(bit, jnp.roll(x, d, axis=2),
                               jnp.roll(x, -d, axis=2))
            else:
                e = d // l
                bit = (riota & e) != 0
                vp = jnp.where(bit, jnp.roll(x, e, axis=1),
                               jnp.roll(x, -e, axis=1))
            want_min = asc != bit
            x = jnp.where(want_min, jnp.minimum(x, vp), jnp.maximum(x, vp))
            d //= 2
        k *= 2
    return x


def _prefix_sum_2d(x, riota, ciota):
    bb, r, l = x.shape
    d = 1
    while d < l:
        x = x + jnp.where(ciota >= d, jnp.roll(x, d, axis=2), 0.0)
        d *= 2
    rowtot = jax.lax.slice(x, (0, 0, l - 1), (bb, r, l))
    t = rowtot
    e = 1
    while e < r:
        t = t + jnp.where(jax.lax.slice(riota, (0, 0, 0), (bb, r, 1)) >= e,
                          jnp.roll(t, e, axis=1), 0.0)
        e *= 2
    return x + (t - rowtot)


def _make_perm_body(nib, ibk, l):
    def _perm_body(s16_ref, minv_ref, c0_ref, u_ref, ru_ref, d_ref,
                   out_ref, v_scr, rv_scr):
        i = pl.program_id(0)
        n = out_ref.shape[1]
        nq = n // ibk

        @pl.when(i == 0)
        def _():
            fill = jnp.min(minv_ref[...]) - 1.0
            s = jnp.where(s16_ref[...] < -1.0e38, fill, s16_ref[...])
            ss = _bitonic_sort_desc(s)
            c0 = c0_ref[...]
            bv = jnp.exp(ss - c0)
            av = 1.0 / bv
            riota = jax.lax.broadcasted_iota(jnp.int32, ss.shape, 1)
            ciota = jax.lax.broadcasted_iota(jnp.int32, ss.shape, 2)
            pa = _prefix_sum_2d(av, riota, ciota)
            pb = _prefix_sum_2d(bv, riota, ciota)
            bt = jnp.sum(bv, axis=(1, 2), keepdims=True)
            denom = bv * pa + av * (bt - pb)
            rd = 1.0 / denom
            v_scr[...] = bv * rd
            rv_scr[...] = av * rd

        @pl.when(i > 0)
        def _():
            t = i - 1
            b = t // nib
            ib = t % nib
            rpb = ibk // l
            v4 = v_scr[pl.ds(b, 1), pl.ds(ib * rpb, rpb), :][0]
            rv4 = rv_scr[pl.ds(b, 1), pl.ds(ib * rpb, rpb), :][0]
            vrow = jnp.concatenate(
                [jax.lax.slice(v4, (tt, 0), (tt + 1, l)) for tt in range(rpb)],
                axis=1)
            rvrow = jnp.concatenate(
                [jax.lax.slice(rv4, (tt, 0), (tt + 1, l)) for tt in range(rpb)],
                axis=1)
            for q in range(nq):
                sl = pl.ds(q * ibk, ibk)

                @pl.when(ib == q)
                def _():
                    eye = (jax.lax.broadcasted_iota(jnp.int32, (ibk, ibk), 0) ==
                           jax.lax.broadcasted_iota(jnp.int32, (ibk, ibk), 1))
                    p = jnp.minimum(u_ref[0, sl, :] * rvrow,
                                    ru_ref[0, sl, :] * vrow)
                    out_ref[0, sl, :] = jnp.where(eye, p + d_ref[0, sl, :], p)

                @pl.when(ib != q)
                def _():
                    out_ref[0, sl, :] = jnp.minimum(u_ref[0, sl, :] * rvrow,
                                                    ru_ref[0, sl, :] * vrow)
    return _perm_body


def kernel(node_features, mask, W, b, noise):
    B, N, D = node_features.shape
    R, L = N // 128, 128
    mask_col = mask.astype(jnp.int32).reshape(B, N, 1)
    b2 = b.reshape(1, 1)

    fcol = pl.BlockSpec((1, N, 1), lambda i: (i, 0, 0))
    fsc = pl.BlockSpec((1, 1, 1), lambda i: (i, 0, 0))
    ocol = jax.ShapeDtypeStruct((B, N, 1), jnp.float32)
    osc = jax.ShapeDtypeStruct((B, 1, 1), jnp.float32)

    s16, minv, c0v, ucol, rucol, dcol = pl.pallas_call(
        _scores_body,
        grid=(B,),
        in_specs=[
            pl.BlockSpec((1, N, D), lambda i: (i, 0, 0)),
            pl.BlockSpec((1, N, D), lambda i: (i, 0, 0)),
            pl.BlockSpec((D, 1), lambda i: (0, 0)),
            pl.BlockSpec((1, 1), lambda i: (0, 0)),
            fcol,
        ],
        out_specs=[pl.BlockSpec((1, R, L), lambda i: (i, 0, 0)),
                   fsc, fsc, fcol, fcol, fcol],
        out_shape=[jax.ShapeDtypeStruct((B, R, L), jnp.float32),
                   osc, osc, ocol, ocol, ocol],
        interpret=_INTERPRET,
    )(node_features, noise, W, b2, mask_col)

    IBK = 2048
    NIB = N // IBK

    def bidx(i):
        return jnp.maximum(i - 1, 0) // NIB

    def ibidx(i):
        return jnp.maximum(i - 1, 0) % NIB

    out = pl.pallas_call(
        _make_perm_body(NIB, IBK, L),
        grid=(1 + B * NIB,),
        in_specs=[
            pl.BlockSpec((B, R, L), lambda i: (0, 0, 0)),
            pl.BlockSpec((B, 1, 1), lambda i: (0, 0, 0)),
            pl.BlockSpec((B, 1, 1), lambda i: (0, 0, 0)),
            pl.BlockSpec((1, N, 1), lambda i: (bidx(i), 0, 0)),
            pl.BlockSpec((1, N, 1), lambda i: (bidx(i), 0, 0)),
            pl.BlockSpec((1, N, 1), lambda i: (bidx(i), 0, 0)),
        ],
        out_specs=pl.BlockSpec((1, N, IBK), lambda i: (bidx(i), 0, ibidx(i))),
        out_shape=jax.ShapeDtypeStruct((B, N, N), jnp.float32),
        scratch_shapes=[
            pltpu.VMEM((B, R, L), jnp.float32),
            pltpu.VMEM((B, R, L), jnp.float32),
        ],
        interpret=_INTERPRET,
    )(s16, minv, c0v, ucol, rucol, dcol)
    return out


# speedup vs baseline: 1.4894x; 1.0031x over previous
import jax
import jax.numpy as jnp
from jax.experimental import pallas as pl
from jax.experimental.pallas import tpu as pltpu


_SENT = -3.0e38


def _scores_body(nf_ref, noise_ref, w_ref, b_ref, m_ref,
                 s16_ref, minv_ref, c0_ref, u_ref, ru_ref, d_ref):
    n = nf_ref.shape[1]
    l = 128
    x = nf_ref[0] + 0.05 * noise_ref[0]
    scol = jax.lax.dot_general(
        x, w_ref[...], (((1,), (0,)), ((), ())),
        preferred_element_type=jnp.float32) + b_ref[0, 0]

    mn = jnp.min(scol)
    c0 = (jnp.max(scol) + mn) * 0.5
    minv_ref[...] = mn.reshape(1, 1, 1)
    c0_ref[...] = c0.reshape(1, 1, 1)

    mf = (m_ref[0] != 0).astype(jnp.float32)
    eb = jnp.exp(scol - c0)
    u_ref[0] = mf * eb
    ru_ref[0] = mf / eb
    d_ref[0] = 1.0 - mf

    smcol = jnp.where(mf > 0.5, scol, _SENT)
    eye128 = (jax.lax.broadcasted_iota(jnp.int32, (l, l), 0) ==
              jax.lax.broadcasted_iota(jnp.int32, (l, l), 1)
              ).astype(jnp.float32)
    rows = []
    for r in range(n // l):
        smr = jax.lax.slice(smcol, (r * l, 0), ((r + 1) * l, 1))
        rows.append(jax.lax.dot_general(
            smr, eye128, (((0,), (0,)), ((), ())),
            preferred_element_type=jnp.float32))
    s16_ref[0] = jnp.concatenate(rows, axis=0)


def _bitonic_sort_desc(x):
    _, r, l = x.shape
    n = r * l
    riota = jax.lax.broadcasted_iota(jnp.int32, x.shape, 1)
    ciota = jax.lax.broadcasted_iota(jnp.int32, x.shape, 2)
    k = 2
    while k <= n:
        asc = ((riota * l + ciota) & k) != 0 if k < n else (ciota < 0)
        d = k // 2
        while d >= 1:
            if d < l:
                bit = (ciota & d) != 0
                vp = jnp.where(bit, jnp.roll(x, d, axis=2),
                               jnp.roll(x, -d, axis=2))
            else:
                e = d // l
                bit = (riota & e) != 0
                vp = jnp.where(bit, jnp.roll(x, e, axis=1),
                               jnp.roll(x, -e, axis=1))
            want_min = asc != bit
            x = jnp.where(want_min, jnp.minimum(x, vp), jnp.maximum(x, vp))
            d //= 2
        k *= 2
    return x


def _prefix_sum_2d(x, riota, ciota):
    bb, r, l = x.shape
    d = 1
    while d < l:
        x = x + jnp.where(ciota >= d, jnp.roll(x, d, axis=2), 0.0)
        d *= 2
    rowtot = jax.lax.slice(x, (0, 0, l - 1), (bb, r, l))
    t = rowtot
    e = 1
    while e < r:
        t = t + jnp.where(jax.lax.slice(riota, (0, 0, 0), (bb, r, 1)) >= e,
                          jnp.roll(t, e, axis=1), 0.0)
        e *= 2
    return x + (t - rowtot)


def _make_perm_body(nib, ibk, l):
    def _perm_body(s16_ref, minv_ref, c0_ref, u_ref, ru_ref, d_ref,
                   out_ref, v_scr, rv_scr):
        i = pl.program_id(0)
        n = out_ref.shape[1]
        nq = n // ibk

        @pl.when(i == 0)
        def _():
            fill = jnp.min(minv_ref[...]) - 1.0
            s = jnp.where(s16_ref[...] < -1.0e38, fill, s16_ref[...])
            ss = _bitonic_sort_desc(s)
            c0 = c0_ref[...]
            bv = jnp.exp(ss - c0)
            av = 1.0 / bv
            riota = jax.lax.broadcasted_iota(jnp.int32, ss.shape, 1)
            ciota = jax.lax.broadcasted_iota(jnp.int32, ss.shape, 2)
            pa = _prefix_sum_2d(av, riota, ciota)
            pb = _prefix_sum_2d(bv, riota, ciota)
            bt = jnp.sum(bv, axis=(1, 2), keepdims=True)
            denom = bv * pa + av * (bt - pb)
            rd = 1.0 / denom
            v_scr[...] = bv * rd
            rv_scr[...] = av * rd

        @pl.when(i > 0)
        def _():
            t = i - 1
            b = t // nib
            ib = t % nib
            rpb = ibk // l
            v4 = v_scr[pl.ds(b, 1), pl.ds(ib * rpb, rpb), :][0]
            rv4 = rv_scr[pl.ds(b, 1), pl.ds(ib * rpb, rpb), :][0]
            vrow = jnp.concatenate(
                [jax.lax.slice(v4, (tt, 0), (tt + 1, l)) for tt in range(rpb)],
                axis=1)
            rvrow = jnp.concatenate(
                [jax.lax.slice(rv4, (tt, 0), (tt + 1, l)) for tt in range(rpb)],
                axis=1)
            for q in range(nq):
                sl = pl.ds(q * ibk, ibk)

                @pl.when(ib == q)
                def _():
                    eye = (jax.lax.broadcasted_iota(jnp.int32, (ibk, ibk), 0) ==
                           jax.lax.broadcasted_iota(jnp.int32, (ibk, ibk), 1))
                    p = jnp.minimum(u_ref[0, sl, :] * rvrow,
                                    ru_ref[0, sl, :] * vrow)
                    out_ref[0, sl, :] = jnp.where(eye, p + d_ref[0, sl, :], p)

                @pl.when(ib != q)
                def _():
                    out_ref[0, sl, :] = jnp.minimum(u_ref[0, sl, :] * rvrow,
                                                    ru_ref[0, sl, :] * vrow)
    return _perm_body


def kernel(node_features, mask, W, b, noise):
    B, N, D = node_features.shape
    R, L = N // 128, 128
    mask_col = mask.astype(jnp.int32).reshape(B, N, 1)
    b2 = b.reshape(1, 1)

    fcol = pl.BlockSpec((1, N, 1), lambda i: (i, 0, 0))
    fsc = pl.BlockSpec((1, 1, 1), lambda i: (i, 0, 0))
    ocol = jax.ShapeDtypeStruct((B, N, 1), jnp.float32)
    osc = jax.ShapeDtypeStruct((B, 1, 1), jnp.float32)

    s16, minv, c0v, ucol, rucol, dcol = pl.pallas_call(
        _scores_body,
        grid=(B,),
        in_specs=[
            pl.BlockSpec((1, N, D), lambda i: (i, 0, 0)),
            pl.BlockSpec((1, N, D), lambda i: (i, 0, 0)),
            pl.BlockSpec((D, 1), lambda i: (0, 0)),
            pl.BlockSpec((1, 1), lambda i: (0, 0)),
            fcol,
        ],
        out_specs=[pl.BlockSpec((1, R, L), lambda i: (i, 0, 0)),
                   fsc, fsc, fcol, fcol, fcol],
        out_shape=[jax.ShapeDtypeStruct((B, R, L), jnp.float32),
                   osc, osc, ocol, ocol, ocol],
    )(node_features, noise, W, b2, mask_col)

    IBK = 2048
    NIB = N // IBK

    def bidx(i):
        return jnp.maximum(i - 1, 0) // NIB

    def ibidx(i):
        return jnp.maximum(i - 1, 0) % NIB

    out = pl.pallas_call(
        _make_perm_body(NIB, IBK, L),
        grid=(1 + B * NIB,),
        in_specs=[
            pl.BlockSpec((B, R, L), lambda i: (0, 0, 0)),
            pl.BlockSpec((B, 1, 1), lambda i: (0, 0, 0)),
            pl.BlockSpec((B, 1, 1), lambda i: (0, 0, 0)),
            pl.BlockSpec((1, N, 1), lambda i: (bidx(i), 0, 0)),
            pl.BlockSpec((1, N, 1), lambda i: (bidx(i), 0, 0)),
            pl.BlockSpec((1, N, 1), lambda i: (bidx(i), 0, 0)),
        ],
        out_specs=pl.BlockSpec((1, N, IBK), lambda i: (bidx(i), 0, ibidx(i))),
        out_shape=jax.ShapeDtypeStruct((B, N, N), jnp.float32),
        scratch_shapes=[
            pltpu.VMEM((B, R, L), jnp.float32),
            pltpu.VMEM((B, R, L), jnp.float32),
        ],
    )(s16, minv, c0v, ucol, rucol, dcol)
    return out
